# baseline (speedup 1.0000x reference)
import functools

import jax
import jax.numpy as jnp
from jax import lax
from jax.experimental import pallas as pl
from jax.experimental.pallas import tpu as pltpu
from jax.experimental.pallas import tpu_sc as plsc

N = 50000
E = 800000
B = 32
HEADS = 4
BLK = 512
GRID = 98
NPAD = GRID * BLK
E2 = E + N
E2PAD = 851968

NEG_INF = -1e30

CN = 4096
SHIFT = 12
NCHUNK = 13
CTOT = NCHUNK * CN

LAYER_GEOM = {
    'g1': (64, 128, 32, 36, 144, 256),
    'g2': (128, 256, 64, 68, 272, 384),
    'g3': (256, 256, 64, 68, 272, 384),
    'g4': (256, 128, 32, 36, 144, 256),
}


_SQRT2_INV = 0.7071067811865476


def _gelu(x):
    return 0.5 * x * (1.0 + lax.erf(x * _SQRT2_INV))


def _ln(x, g, b, eps=1e-5):
    m = jnp.mean(x, axis=-1, keepdims=True)
    v = jnp.mean((x - m) ** 2, axis=-1, keepdims=True)
    return (x - m) / jnp.sqrt(v + eps) * g + b


def _asd_out(hb, Ss, Sdrev):
    blk = hb.shape[0]
    a_s = jax.lax.dot_general(hb, Ss, (((1,), (0,)), ((), ())))
    a_dr = jax.lax.dot_general(hb, Sdrev, (((1,), (0,)), ((), ())))
    z = jnp.zeros((blk, 8), jnp.float32)
    return jnp.concatenate([a_s, z, a_dr], axis=1)


def _proj_outputs(hb, c, sec, rowp, asd):
    pieces = []
    blk = hb.shape[0]
    rowf = 4 * sec
    one = jnp.ones((blk, 1), jnp.float32)
    zer = jnp.zeros((blk, sec - c - 1), jnp.float32)
    for h in range(HEADS):
        pieces.append(hb[:, h * c:(h + 1) * c])
        pieces.append(one)
        if sec - c - 1:
            pieces.append(zer)
    pieces.append(jnp.zeros((blk, rowp - rowf - 16), jnp.float32))
    pieces.append(asd)
    return jnp.concatenate(pieces, axis=1)


def _stage_a_body(x_ref, neW, neb, neg, nebe, W1, Ss, Sd,
                  h0_ref, hext_ref, asd_ref):
    xb = x_ref[...]
    t = jax.lax.dot_general(xb, neW[...], (((1,), (0,)), ((), ()))) + neb[...]
    h0 = _gelu(_ln(t, neg[...], nebe[...]))
    h0_ref[...] = h0
    hb = jax.lax.dot_general(h0, W1[...], (((1,), (0,)), ((), ())))
    asd = _asd_out(hb, Ss[...], Sd[...])
    hext_ref[...] = _proj_outputs(hb, 32, 36, 256, asd)
    asd_ref[...] = asd


def _stage_a(x_p, p, Ss, Sd):
    full = lambda a: pl.BlockSpec(a.shape, lambda i: (0,) * a.ndim)
    return pl.pallas_call(
        _stage_a_body,
        grid=(GRID,),
        in_specs=[pl.BlockSpec((BLK, 16), lambda i: (i, 0)),
                  full(p['ne_W']), full(p['ne_b2']), full(p['ne_g2']),
                  full(p['ne_be2']), full(p['g1_W']), full(Ss), full(Sd)],
        out_specs=[pl.BlockSpec((BLK, 64), lambda i: (i, 0)),
                   pl.BlockSpec((BLK, 256), lambda i: (i, 0)),
                   pl.BlockSpec((BLK, 16), lambda i: (i, 0))],
        out_shape=[jax.ShapeDtypeStruct((NPAD, 64), jnp.float32),
                   jax.ShapeDtypeStruct((NPAD, 256), jnp.float32),
                   jax.ShapeDtypeStruct((NPAD, 16), jnp.float32)],
    )(x_p, p['ne_W'], p['ne_b2'], p['ne_g2'], p['ne_be2'], p['g1_W'], Ss, Sd)


def _finalize_body(c, sec, dout, has_res, has_next, c2, sec2, rowp2,
                   acc_ref, hprev_ref, bias, lng, lnb, rW, rb, Wn, Ssn, Sdn,
                   hout_ref, hextn_ref, asdn_ref):
    acc = acc_ref[...]
    outs = []
    for h in range(HEADS):
        num = acc[:, h * sec:h * sec + c]
        den = acc[:, h * sec + c:h * sec + c + 1]
        outs.append(num / (den + 1e-16))
    gat = jnp.concatenate(outs, axis=1) + bias[...]
    gat = _ln(gat, lng[...], lnb[...])
    hp = hprev_ref[...]
    if has_res:
        res = jax.lax.dot_general(hp, rW[...], (((1,), (0,)), ((), ()))) + rb[...]
    else:
        res = hp
    hout = _gelu(gat + res)
    hout_ref[...] = hout
    if has_next:
        hb = jax.lax.dot_general(hout, Wn[...], (((1,), (0,)), ((), ())))
        asdn = _asd_out(hb, Ssn[...], Sdn[...])
        hextn_ref[...] = _proj_outputs(hb, c2, sec2, rowp2, asdn)
        asdn_ref[...] = asdn


def _finalize(name, acc, hprev, p, nxt, Ssn, Sdn):
    din, dout, c, sec, rowf, rowp = LAYER_GEOM[name]
    has_res = (name + '_rW') in p
    has_next = nxt is not None
    if has_next:
        _, dn, c2, sec2, rowf2, rowp2 = LAYER_GEOM[nxt]
    else:
        dn, c2, sec2, rowf2, rowp2 = 8, 1, 2, 8, 128
    rW = p.get(name + '_rW', p[name + '_lng2'])
    rb = p.get(name + '_rb2', p[name + '_lnb2'])
    Wn = p[nxt + '_W'] if has_next else p[name + '_lng2']
    Ssn = Ssn if has_next else p[name + '_lnb2']
    Sdn = Sdn if has_next else p[name + '_lnb2']
    full = lambda a: pl.BlockSpec(a.shape, lambda i: (0,) * a.ndim)
    body = functools.partial(_finalize_body, c, sec, dout, has_res, has_next,
                             c2, sec2, rowp2)
    out_specs = [pl.BlockSpec((BLK, dout), lambda i: (i, 0))]
    out_shape = [jax.ShapeDtypeStruct((NPAD, dout), jnp.float32)]
    if has_next:
        out_specs += [pl.BlockSpec((BLK, rowp2), lambda i: (i, 0)),
                      pl.BlockSpec((BLK, 16), lambda i: (i, 0))]
        out_shape += [jax.ShapeDtypeStruct((NPAD, rowp2), jnp.float32),
                      jax.ShapeDtypeStruct((NPAD, 16), jnp.float32)]
    else:
        out_specs += [pl.BlockSpec((BLK, 8), lambda i: (i, 0)),
                      pl.BlockSpec((BLK, 16), lambda i: (i, 0))]
        out_shape += [jax.ShapeDtypeStruct((NPAD, 8), jnp.float32),
                      jax.ShapeDtypeStruct((NPAD, 16), jnp.float32)]
    res = pl.pallas_call(
        body,
        grid=(GRID,),
        in_specs=[pl.BlockSpec((BLK, rowp), lambda i: (i, 0)),
                  pl.BlockSpec((BLK, din), lambda i: (i, 0)),
                  full(p[name + '_b2']), full(p[name + '_lng2']),
                  full(p[name + '_lnb2']), full(rW), full(rb), full(Wn),
                  full(Ssn), full(Sdn)],
        out_specs=out_specs,
        out_shape=out_shape,
    )(acc[:NPAD], hprev, p[name + '_b2'], p[name + '_lng2'],
      p[name + '_lnb2'], rW, rb, Wn, Ssn, Sdn)
    return res


def _pool_body(h_ref, b_ref, gsum_ref, gmax_ref, gcnt_ref,
               s_sum, s_max, s_cnt):
    i = pl.program_id(0)

    @pl.when(i == 0)
    def _():
        s_sum[...] = jnp.zeros_like(s_sum)
        s_max[...] = jnp.full_like(s_max, NEG_INF)
        s_cnt[...] = jnp.zeros_like(s_cnt)

    hb = h_ref[...]
    bv = b_ref[...].reshape(BLK, 1)
    seg = jax.lax.broadcasted_iota(jnp.int32, (1, B), 1)
    oh = (bv == seg)
    ohf = oh.astype(jnp.float32)
    s_sum[...] += jax.lax.dot_general(ohf, hb, (((0,), (0,)), ((), ())))
    s_cnt[...] += jnp.sum(ohf, axis=0, keepdims=True)
    cur = s_max[...]
    news = []
    for bb in range(B):
        colmask = oh[:, bb:bb + 1]
        vals = jnp.where(colmask, hb, NEG_INF)
        news.append(jnp.max(vals, axis=0, keepdims=True))
    s_max[...] = jnp.maximum(cur, jnp.concatenate(news, axis=0))

    @pl.when(i == GRID - 1)
    def _():
        gsum_ref[...] = s_sum[...]
        gmax_ref[...] = s_max[...]
        gcnt_ref[...] = s_cnt[...]


def _pool(h4, batch3):
    return pl.pallas_call(
        _pool_body,
        grid=(GRID,),
        in_specs=[pl.BlockSpec((BLK, 128), lambda i: (i, 0)),
                  pl.BlockSpec((1, 1, BLK), lambda i: (i, 0, 0))],
        out_specs=[pl.BlockSpec((B, 128), lambda i: (0, 0)),
                   pl.BlockSpec((B, 128), lambda i: (0, 0)),
                   pl.BlockSpec((1, B), lambda i: (0, 0))],
        out_shape=[jax.ShapeDtypeStruct((B, 128), jnp.float32),
                   jax.ShapeDtypeStruct((B, 128), jnp.float32),
                   jax.ShapeDtypeStruct((1, B), jnp.float32)],
        scratch_shapes=[pltpu.VMEM((B, 128), jnp.float32),
                        pltpu.VMEM((B, 128), jnp.float32),
                        pltpu.VMEM((1, B), jnp.float32)],
    )(h4, batch3)


def _tail_body(gsum_ref, gmax_ref, gcnt_ref, u_ref, *refs):
    (ge1W, ge1b, ge1g, ge1be, ge2W, ge2b, ge2g, ge2be,
     ge3W, ge3b, ge3g, ge3be, f1W, f1b, f1g, f1be,
     f2W, f2b, f2g, f2be, f3W, f3b, f3g, f3be,
     c1W, c1b, c2W, c2b, out_ref) = refs
    gsum = gsum_ref[...]
    gmax = gmax_ref[...]
    cnt = gcnt_ref[...].reshape(B, 1)
    gmean = gsum / jnp.maximum(cnt, 1.0)
    gmax = jnp.where(gmax > NEG_INF * 0.5, gmax, 0.0)
    gadd = gsum / 10.0
    gf = jnp.concatenate([gmean, gmax, gadd], axis=1)

    def mlp(xx, W, bb, g, be):
        t = jax.lax.dot_general(xx, W[...], (((1,), (0,)), ((), ()))) + bb[...]
        return _gelu(_ln(t, g[...], be[...]))

    g = u_ref[...]
    g = mlp(g, ge1W, ge1b, ge1g, ge1be)
    g = mlp(g, ge2W, ge2b, ge2g, ge2be)
    g = mlp(g, ge3W, ge3b, ge3g, ge3be)
    comb = jnp.concatenate([gf, g], axis=1)
    comb = mlp(comb, f1W, f1b, f1g, f1be)
    comb = mlp(comb, f2W, f2b, f2g, f2be)
    comb = mlp(comb, f3W, f3b, f3g, f3be)
    h2 = _gelu(jax.lax.dot_general(comb, c1W[...], (((1,), (0,)), ((), ()))) + c1b[...])
    lg = jax.lax.dot_general(h2, c2W[...], (((1,), (0,)), ((), ()))) + c2b[...]
    out_ref[...] = jnp.pad(lg, ((0, 0), (0, 126)))


def _tail(gsum, gmax, gcnt, u, p):
    names = []
    for nm in ['ge1', 'ge2', 'ge3', 'f1', 'f2', 'f3']:
        names += [p[nm + '_W'], p[nm + '_b2'], p[nm + '_g2'], p[nm + '_be2']]
    names += [p['c1_W'], p['c1_b2'], p['c2_W'], p['c2_b2']]
    out = pl.pallas_call(
        _tail_body,
        out_shape=jax.ShapeDtypeStruct((B, 128), jnp.float32),
    )(gsum, gmax, gcnt, u, *names)
    return out[:, :2]


EB = 1024
EPT = E2PAD // 16
NBLK_SC = EPT // EB


def _sc_mesh():
    return plsc.VectorSubcoreMesh(core_axis_name="c", subcore_axis_name="s")


def _take16(x, idx):
    dn = lax.GatherDimensionNumbers(
        offset_dims=(), collapsed_slice_dims=(0,), start_index_map=(0,))
    return lax.gather(x, idx[:, None], dn, (1,),
                      mode=lax.GatherScatterMode.PROMISE_IN_BOUNDS)


def _tree_sum16(x, iota):
    for k in (1, 2, 4, 8):
        x = x + _take16(x, iota ^ k)
    return x


def _tree_min16(x, iota):
    for k in (1, 2, 4, 8):
        x = jnp.minimum(x, _take16(x, iota ^ k))
    return x


CP = 256
SHIFTP = 8
NCHUNKP = CTOT // CP
TRIPS = (NCHUNKP + 31) // 32
NBLKP = E2PAD // EB


def _sc_scatter(name, src2, dst2, asd, hext):
    din, dout, c, sec, rowf, rowp = LAYER_GEOM[name]
    nj = rowf // 16
    njp = rowp // 16
    ZR = 16

    def body(src_hbm, dst_hbm, asd_hbm, hext_hbm, out_hbm,
             sbuf, dbuf, csd, sidx, didx, mbuf, pbuf,
             gidx, arows, grow, accp, sem, sem2):
        cid = lax.axis_index("c")
        sid = lax.axis_index("s")
        gid = cid * 16 + sid
        iota = lax.iota(jnp.int32, 16)
        zf = jnp.zeros((16,), jnp.float32)
        zi = jnp.zeros((16,), jnp.int32)

        tv = jnp.clip(NCHUNKP - gid * TRIPS, 0, TRIPS)

        def chunk_body(t, _):
            sc = gid * TRIPS + t
            base = sc * CP

            def zacc(z, _2):
                for j in range(njp):
                    accp[z, pl.ds(j * 16, 16)] = zf
                return 0

            lax.fori_loop(0, CP, zacc, 0)

            def blk(b, _2):
                off = b * EB
                pltpu.sync_copy(src_hbm.at[pl.ds(off, EB)], sbuf)
                pltpu.sync_copy(dst_hbm.at[pl.ds(off, EB)], dbuf)

                def cvec(v, pos):
                    sl = pl.ds(v * 16, 16)
                    dv = dbuf[sl]
                    sv = sbuf[sl]
                    m = lax.shift_right_logical(dv, SHIFTP) == sc
                    packed = (lax.shift_left(sv, 13)
                              | jnp.where(dv >= N, 4096, 0)
                              | (dv & (CP - 1)))
                    cnt = _tree_sum16(jnp.where(m, 1, 0), iota)[0]
                    mbuf[...] = jnp.where(m, 1, 0)
                    pbuf[...] = zi

                    def peel(j, _4):
                        mm = mbuf[...]
                        f = _tree_min16(jnp.where(mm != 0, iota, 16), iota)
                        pbuf[...] = jnp.where(iota == j, f, pbuf[...])
                        mbuf[...] = jnp.where(iota == f, 0, mm)
                        return 0

                    lax.fori_loop(0, cnt, peel, 0)
                    comp = _take16(packed, pbuf[...])
                    csd[pl.ds(pos, 16)] = comp
                    return pos + cnt

                pos = lax.fori_loop(0, EB // 16, cvec, 0)
                csd[pl.ds(pos, 16)] = jnp.full((16,), 4096, jnp.int32)
                nb = (pos + 15) // 16

                def gb(g, _3):
                    vsv = csd[pl.ds(g * 16, 16)]
                    dlv = vsv & 8191
                    sidx[...] = lax.shift_right_logical(vsv, 13)
                    dloc = dlv & (CP - 1)
                    didx[...] = dloc
                    gidx[...] = dloc + base
                    cp = pltpu.async_copy(hext_hbm.at[sidx], grow, sem)
                    cp2 = pltpu.async_copy(asd_hbm.at[gidx], arows, sem2)
                    cp.wait()
                    cp2.wait()
                    for rr in range(16):
                        arow = grow[rr, pl.ds(rowp - 16, 16)]
                        dl = dlv[rr]
                        drow = arows[rr, pl.ds(0, 16)]
                        w = arow + lax.rev(drow, (0,))
                        al = jnp.where(w >= 0.0, w, 0.2 * w)
                        er = jnp.exp(al) * jnp.where(dl >= 4096, 0.0, 1.0)
                        eb = [jnp.full((16,), er[h], jnp.float32)
                              for h in range(HEADS)]
                        dr = dl & (CP - 1)
                        for j in range(nj):
                            col = iota + j * 16
                            exv = eb[0]
                            for k in range(1, HEADS):
                                exv = jnp.where(col >= k * sec, eb[k], exv)
                            sl = pl.ds(j * 16, 16)
                            accp[dr, sl] = accp[dr, sl] + grow[rr, sl] * exv
                    return 0

                lax.fori_loop(0, nb, gb, 0)
                return 0

            lax.fori_loop(0, NBLKP, blk, 0)
            pltpu.sync_copy(accp, out_hbm.at[pl.ds(base, CP)])
            return 0

        lax.fori_loop(0, tv, chunk_body, 0)

    f = pl.kernel(
        body,
        out_type=jax.ShapeDtypeStruct((CTOT, rowp), jnp.float32),
        mesh=_sc_mesh(),
        scratch_types=[
            pltpu.VMEM((EB,), jnp.int32),
            pltpu.VMEM((EB,), jnp.int32),
            pltpu.VMEM((EB + 16,), jnp.int32),
            pltpu.VMEM((16,), jnp.int32),
            pltpu.VMEM((16,), jnp.int32),
            pltpu.VMEM((16,), jnp.int32),
            pltpu.VMEM((16,), jnp.int32),
            pltpu.VMEM((16,), jnp.int32),
            pltpu.VMEM((16, 128), jnp.float32),
            pltpu.VMEM((16, rowp), jnp.float32),
            pltpu.VMEM((CP, rowp), jnp.float32),
            pltpu.SemaphoreType.DMA,
            pltpu.SemaphoreType.DMA,
        ],
    )
    return f(src2, dst2, asd, hext)


def _block_diag_att(att, reverse=False):
    heads, c = att.shape
    S = jnp.zeros((heads * c, 4), jnp.float32)
    for h in range(heads):
        col = (heads - 1 - h) if reverse else h
        S = S.at[h * c:(h + 1) * c, col].set(att[h])
    return S


def kernel(x, edge_index, batch, u, params):
    p = dict(params)
    for k in list(p.keys()):
        v = p[k]
        if v.ndim == 1:
            p[k + '2'] = v.reshape(1, -1)
    Ss = {}
    Sd = {}
    for nm in ['g1', 'g2', 'g3', 'g4']:
        Ss[nm] = _block_diag_att(p[nm + '_as'])
        Sd[nm] = _block_diag_att(p[nm + '_ad'], reverse=True)

    loop = jnp.arange(N, dtype=edge_index.dtype)
    pad_src = jnp.zeros((E2PAD - E2,), edge_index.dtype)
    pad_dst = jnp.full((E2PAD - E2,), N, edge_index.dtype)
    src2 = jnp.concatenate([edge_index[0], loop, pad_src])
    dst2 = jnp.concatenate([edge_index[1], loop, pad_dst])

    x_p = jnp.zeros((NPAD, 16), jnp.float32).at[:N].set(x)
    batch3 = jnp.full((NPAD,), B, jnp.int32).at[:N].set(batch).reshape(GRID, 1, BLK)

    pad_asd = lambda a: jnp.zeros((CTOT, 128), jnp.float32).at[:NPAD, :16].set(a)

    h0, hext1, asd1 = _stage_a(x_p, p, Ss['g1'], Sd['g1'])

    acc1 = _sc_scatter('g1', src2, dst2, pad_asd(asd1), hext1)
    h1, hext2, asd2 = _finalize('g1', acc1, h0, p, 'g2', Ss['g2'], Sd['g2'])
    acc2 = _sc_scatter('g2', src2, dst2, pad_asd(asd2), hext2)
    h2, hext3, asd3 = _finalize('g2', acc2, h1, p, 'g3', Ss['g3'], Sd['g3'])
    acc3 = _sc_scatter('g3', src2, dst2, pad_asd(asd3), hext3)
    h3, hext4, asd4 = _finalize('g3', acc3, h2, p, 'g4', Ss['g4'], Sd['g4'])
    acc4 = _sc_scatter('g4', src2, dst2, pad_asd(asd4), hext4)
    h4, _, _ = _finalize('g4', acc4, h3, p, None, None, None)

    gsum, gmax, gcnt = _pool(h4, batch3)
    return _tail(gsum, gmax, gcnt, u, p)
